# Initial kernel scaffold; baseline (speedup 1.0000x reference)
import functools

import jax
import jax.numpy as jnp
from jax import lax
from jax.experimental import pallas as pl
from jax.experimental.pallas import tpu as pltpu
from jax.experimental.pallas import tpu_sc as plsc

BATCH = 16384
EMB = 128

_INFO = plsc.get_sparse_core_info()
_NC, _NS = _INFO.num_cores, _INFO.num_subcores
_NW = _NC * _NS
_BPW = BATCH // _NW
_CH = 256
_NCH = _BPW // _CH

_mesh = plsc.VectorSubcoreMesh(core_axis_name="c", subcore_axis_name="s")


@functools.partial(
    pl.kernel,
    mesh=_mesh,
    out_type=[jax.ShapeDtypeStruct((BATCH, EMB), jnp.float32)] * 4,
    scratch_types=[
        pltpu.VMEM((_NCH, _CH), jnp.int32),
        pltpu.VMEM((_NCH, _CH), jnp.int32),
        pltpu.VMEM((_CH, EMB), jnp.float32),
        pltpu.VMEM((_CH, EMB), jnp.float32),
        pltpu.SemaphoreType.DMA,
        pltpu.SemaphoreType.DMA,
    ],
)
def _gather4(uidx_hbm, midx_hbm, ug_t, mg_t, um_t, mm_t,
             ug_o, mg_o, um_o, mm_o,
             uvec, mvec, bufa, bufb, sema, semb):
    wid = lax.axis_index("s") * _NC + lax.axis_index("c")
    base = wid * _BPW
    for c in range(_NCH):
        pltpu.sync_copy(uidx_hbm.at[pl.ds(base + c * _CH, _CH)], uvec.at[c])
        pltpu.sync_copy(midx_hbm.at[pl.ds(base + c * _CH, _CH)], mvec.at[c])

    jobs = []
    for tab, ivec, out in ((ug_t, uvec, ug_o), (mg_t, mvec, mg_o),
                           (um_t, uvec, um_o), (mm_t, mvec, mm_o)):
        for c in range(_NCH):
            jobs.append((tab, ivec, out, c))

    bufs = (bufa, bufb)
    sems = (sema, semb)
    for j, (tab, ivec, out, c) in enumerate(jobs):
        b = j % 2
        pltpu.async_copy(tab.at[ivec.at[c]], bufs[b], sems[b])
        if j >= 1:
            pb = (j - 1) % 2
            ptab, pivec, pout, pc = jobs[j - 1]
            pltpu.make_async_copy(ptab.at[pivec.at[pc]], bufs[pb], sems[pb]).wait()
            pltpu.sync_copy(bufs[pb], pout.at[pl.ds(base + pc * _CH, _CH)])
    j = len(jobs) - 1
    b = j % 2
    tab, ivec, out, c = jobs[j]
    pltpu.make_async_copy(tab.at[ivec.at[c]], bufs[b], sems[b]).wait()
    pltpu.sync_copy(bufs[b], out.at[pl.ds(base + c * _CH, _CH)])


_BM = 2048


def _tc_body(ug, mg, um, mm, gmfw, w0a, w0b, b0, w1, b1, w2, b2, w3, b3,
             fmw, cconst, out_ref):
    g = jnp.sum(ug[...] * mg[...] * gmfw[...], axis=1, keepdims=True)
    h = jnp.maximum(
        jnp.dot(um[...], w0a[...], preferred_element_type=jnp.float32)
        + jnp.dot(mm[...], w0b[...], preferred_element_type=jnp.float32)
        + b0[...], 0.0)
    h = jnp.maximum(jnp.dot(h, w1[...], preferred_element_type=jnp.float32) + b1[...], 0.0)
    h = jnp.maximum(jnp.dot(h, w2[...], preferred_element_type=jnp.float32) + b2[...], 0.0)
    h = jnp.maximum(jnp.dot(h, w3[...], preferred_element_type=jnp.float32) + b3[...], 0.0)
    m = jnp.dot(h, fmw[...], preferred_element_type=jnp.float32)
    out_ref[...] = g + m + cconst[...]


def _full(shape):
    return pl.BlockSpec(shape, lambda i: (0, 0))


_tc_call = pl.pallas_call(
    _tc_body,
    grid=(BATCH // _BM,),
    in_specs=[
        pl.BlockSpec((_BM, EMB), lambda i: (i, 0)),
        pl.BlockSpec((_BM, EMB), lambda i: (i, 0)),
        pl.BlockSpec((_BM, EMB), lambda i: (i, 0)),
        pl.BlockSpec((_BM, EMB), lambda i: (i, 0)),
        _full((1, EMB)),
        _full((EMB, 64)),
        _full((EMB, 64)),
        _full((1, 64)),
        _full((64, 32)),
        _full((1, 32)),
        _full((32, 16)),
        _full((1, 16)),
        _full((16, 8)),
        _full((1, 8)),
        _full((8, 1)),
        _full((1, 1)),
    ],
    out_specs=pl.BlockSpec((_BM, 1), lambda i: (i, 0)),
    out_shape=jax.ShapeDtypeStruct((BATCH, 1), jnp.float32),
)


def kernel(X, user_emb_gmf, movie_emb_gmf, user_emb_mlp, movie_emb_mlp,
           gmf_w, gmf_b, final_mlp_w, final_mlp_b, final_w, final_b,
           mlp_w0, mlp_b0, mlp_w1, mlp_b1, mlp_w2, mlp_b2, mlp_w3, mlp_b3):
    user = X[:, 0]
    movie = X[:, 1]
    ug, mg, um, mm = _gather4(user, movie, user_emb_gmf, movie_emb_gmf,
                              user_emb_mlp, movie_emb_mlp)
    fw0 = final_w[0, 0]
    fw1 = final_w[1, 0]
    gmfw = (gmf_w[:, 0] * fw0).reshape(1, EMB)
    fmw = final_mlp_w * fw1
    cconst = (final_b[0] + fw0 * gmf_b[0] + fw1 * final_mlp_b[0]).reshape(1, 1)
    return _tc_call(ug, mg, um, mm, gmfw,
                    mlp_w0[:EMB], mlp_w0[EMB:], mlp_b0.reshape(1, -1),
                    mlp_w1, mlp_b1.reshape(1, -1),
                    mlp_w2, mlp_b2.reshape(1, -1),
                    mlp_w3, mlp_b3.reshape(1, -1),
                    fmw, cconst)


# speedup vs baseline: 4.2379x; 4.2379x over previous
import functools

import jax
import jax.numpy as jnp
from jax import lax
from jax.experimental import pallas as pl
from jax.experimental.pallas import tpu as pltpu
from jax.experimental.pallas import tpu_sc as plsc

BATCH = 16384
EMB = 128

_INFO = plsc.get_sparse_core_info()
_NC, _NS = _INFO.num_cores, _INFO.num_subcores
_NW = _NC * _NS
_BPW = BATCH // _NW
_CH = 128
_NCH = _BPW // _CH

_mesh = plsc.VectorSubcoreMesh(core_axis_name="c", subcore_axis_name="s")


@functools.partial(
    pl.kernel,
    mesh=_mesh,
    out_type=[jax.ShapeDtypeStruct((BATCH, EMB), jnp.float32)] * 4,
    scratch_types=[
        pltpu.VMEM((_NCH, _CH), jnp.int32),
        pltpu.VMEM((_NCH, _CH), jnp.int32),
        pltpu.VMEM((_CH, EMB), jnp.float32),
        pltpu.VMEM((_CH, EMB), jnp.float32),
        pltpu.SemaphoreType.DMA,
        pltpu.SemaphoreType.DMA,
    ],
)
def _gather4(uidx_hbm, midx_hbm, ug_t, mg_t, um_t, mm_t,
             ug_o, mg_o, um_o, mm_o,
             uvec, mvec, bufa, bufb, sema, semb):
    wid = lax.axis_index("s") * _NC + lax.axis_index("c")
    base = wid * _BPW
    for c in range(_NCH):
        pltpu.sync_copy(uidx_hbm.at[pl.ds(base + c * _CH, _CH)], uvec.at[c])
        pltpu.sync_copy(midx_hbm.at[pl.ds(base + c * _CH, _CH)], mvec.at[c])

    jobs = []
    for tab, ivec, out in ((ug_t, uvec, ug_o), (mg_t, mvec, mg_o),
                           (um_t, uvec, um_o), (mm_t, mvec, mm_o)):
        for c in range(_NCH):
            jobs.append((tab, ivec, out, c))

    bufs = (bufa, bufb)
    sems = (sema, semb)
    for j, (tab, ivec, out, c) in enumerate(jobs):
        b = j % 2
        pltpu.async_copy(tab.at[ivec.at[c]], bufs[b], sems[b])
        if j >= 1:
            pb = (j - 1) % 2
            ptab, pivec, pout, pc = jobs[j - 1]
            pltpu.make_async_copy(ptab.at[pivec.at[pc]], bufs[pb], sems[pb]).wait()
            pltpu.sync_copy(bufs[pb], pout.at[pl.ds(base + pc * _CH, _CH)])
    j = len(jobs) - 1
    b = j % 2
    tab, ivec, out, c = jobs[j]
    pltpu.make_async_copy(tab.at[ivec.at[c]], bufs[b], sems[b]).wait()
    pltpu.sync_copy(bufs[b], out.at[pl.ds(base + c * _CH, _CH)])


_BM = 2048


def _tc_body(ug, mg, um, mm, gmfw, w0a, w0b, b0, w1, b1, w2, b2, w3, b3,
             fmw, cconst, out_ref):
    g = jnp.sum(ug[...] * mg[...] * gmfw[...], axis=1, keepdims=True)
    h = jnp.maximum(
        jnp.dot(um[...], w0a[...], preferred_element_type=jnp.float32)
        + jnp.dot(mm[...], w0b[...], preferred_element_type=jnp.float32)
        + b0[...], 0.0)
    h = jnp.maximum(jnp.dot(h, w1[...], preferred_element_type=jnp.float32) + b1[...], 0.0)
    h = jnp.maximum(jnp.dot(h, w2[...], preferred_element_type=jnp.float32) + b2[...], 0.0)
    h = jnp.maximum(jnp.dot(h, w3[...], preferred_element_type=jnp.float32) + b3[...], 0.0)
    m = jnp.dot(h, fmw[...], preferred_element_type=jnp.float32)
    out_ref[...] = g + m + cconst[...]


def _full(shape):
    return pl.BlockSpec(shape, lambda i: (0, 0))


_tc_call = pl.pallas_call(
    _tc_body,
    grid=(BATCH // _BM,),
    in_specs=[
        pl.BlockSpec((_BM, EMB), lambda i: (i, 0)),
        pl.BlockSpec((_BM, EMB), lambda i: (i, 0)),
        pl.BlockSpec((_BM, EMB), lambda i: (i, 0)),
        pl.BlockSpec((_BM, EMB), lambda i: (i, 0)),
        _full((1, EMB)),
        _full((EMB, 64)),
        _full((EMB, 64)),
        _full((1, 64)),
        _full((64, 32)),
        _full((1, 32)),
        _full((32, 16)),
        _full((1, 16)),
        _full((16, 8)),
        _full((1, 8)),
        _full((8, 1)),
        _full((1, 1)),
    ],
    out_specs=pl.BlockSpec((_BM, 1), lambda i: (i, 0)),
    out_shape=jax.ShapeDtypeStruct((BATCH, 1), jnp.float32),
)


def kernel(X, user_emb_gmf, movie_emb_gmf, user_emb_mlp, movie_emb_mlp,
           gmf_w, gmf_b, final_mlp_w, final_mlp_b, final_w, final_b,
           mlp_w0, mlp_b0, mlp_w1, mlp_b1, mlp_w2, mlp_b2, mlp_w3, mlp_b3):
    user = X[:, 0]
    movie = X[:, 1]
    ug, mg, um, mm = _gather4(user, movie, user_emb_gmf, movie_emb_gmf,
                              user_emb_mlp, movie_emb_mlp)
    fw0 = final_w[0, 0]
    fw1 = final_w[1, 0]
    gmfw = (gmf_w[:, 0] * fw0).reshape(1, EMB)
    fmw = final_mlp_w * fw1
    cconst = (final_b[0] + fw0 * gmf_b[0] + fw1 * final_mlp_b[0]).reshape(1, 1)
    return _tc_call(ug, mg, um, mm, gmfw,
                    mlp_w0[:EMB], mlp_w0[EMB:], mlp_b0.reshape(1, -1),
                    mlp_w1, mlp_b1.reshape(1, -1),
                    mlp_w2, mlp_b2.reshape(1, -1),
                    mlp_w3, mlp_b3.reshape(1, -1),
                    fmw, cconst)
